# Initial kernel scaffold; baseline (speedup 1.0000x reference)
import jax
import jax.numpy as jnp
from jax.experimental import pallas as pl


def kernel(x, edge_index, batch, fingerprint, c1W1, c1b1, c1W2, c1b2, g1, be1, c2W1, c2b1, c2W2, c2b2, g2, be2, c3W1, c3b1, c3W2, c3b2, g3, be3, oW1, ob1, og, obe, oW3, ob3):
    raise NotImplementedError("write your pallas kernel here")


# speedup vs baseline: 3.8116x; 3.8116x over previous
import functools

import jax
import jax.numpy as jnp
from jax import lax
from jax.experimental import pallas as pl
from jax.experimental.pallas import tpu as pltpu
from jax.experimental.pallas import tpu_sc as plsc

_NC = 2
_NS = 16
_NW = _NC * _NS
_CHUNK = 128


def _sc_segsum(h, src3, dst3, zrows, n_pad):
    N, D = h.shape
    nch = src3.shape[1]
    rows_per_tile = n_pad // _NS
    mesh = plsc.VectorSubcoreMesh(core_axis_name="c", subcore_axis_name="s")

    @functools.partial(
        pl.kernel,
        out_type=jax.ShapeDtypeStruct((_NC, n_pad, D), jnp.float32),
        mesh=mesh,
        scratch_types=[
            pltpu.VMEM((nch, _CHUNK), jnp.int32),
            pltpu.VMEM((nch, _CHUNK), jnp.int32),
            pltpu.VMEM((_CHUNK, D), jnp.float32),
            pltpu.VMEM_SHARED((n_pad, D), jnp.float32),
            pltpu.SemaphoreType.DMA,
        ],
    )
    def seg_kernel(h_hbm, src_hbm, dst_hbm, z_hbm, out_hbm,
                   src_v, dst_v, buf, acc, sem):
        cid = lax.axis_index("c")
        sid = lax.axis_index("s")
        wid = cid * _NS + sid
        row0 = sid * rows_per_tile
        pltpu.sync_copy(z_hbm, acc.at[pl.ds(row0, rows_per_tile)])
        pltpu.sync_copy(src_hbm.at[wid], src_v)
        pltpu.sync_copy(dst_hbm.at[wid], dst_v)
        plsc.subcore_barrier()

        def body(j, carry):
            pltpu.async_copy(h_hbm.at[src_v.at[j]], buf, sem).wait()
            pltpu.sync_copy(buf, acc.at[dst_v.at[j]], add=True)
            return carry

        lax.fori_loop(0, nch, body, 0)
        plsc.subcore_barrier()
        pltpu.sync_copy(acc.at[pl.ds(row0, rows_per_tile)],
                        out_hbm.at[cid, pl.ds(row0, rows_per_tile)])

    return seg_kernel(h, src3, dst3, zrows)


def _tc_mlp_stats(x, parts, W1, b1, W2, b2, g, be, blk=1000):
    N, D = x.shape
    hmid = W1.shape[1]
    h2 = W2.shape[1]
    nblk = N // blk

    def body(x_ref, p0_ref, p1_ref, w1_ref, b1_ref, w2_ref, b2_ref,
             g_ref, be_ref, y_ref, aff_ref, s_ref, q_ref):
        i = pl.program_id(0)
        h0 = x_ref[...] + p0_ref[...] + p1_ref[...]
        a = jnp.maximum(
            jnp.dot(h0.astype(jnp.bfloat16), w1_ref[...].astype(jnp.bfloat16),
                    preferred_element_type=jnp.float32) + b1_ref[...], 0.0)
        y = (jnp.dot(a.astype(jnp.bfloat16), w2_ref[...].astype(jnp.bfloat16),
                     preferred_element_type=jnp.float32) + b2_ref[...])
        y_ref[...] = y

        @pl.when(i == 0)
        def _():
            s_ref[...] = jnp.zeros_like(s_ref)
            q_ref[...] = jnp.zeros_like(q_ref)

        s_ref[...] += jnp.sum(y, axis=0, keepdims=True)
        q_ref[...] += jnp.sum(y * y, axis=0, keepdims=True)

        @pl.when(i == nblk - 1)
        def _():
            m = s_ref[...] / N
            v = q_ref[...] / N - m * m
            scale = g_ref[...] / jnp.sqrt(v + 1e-5)
            aff_ref[0:1, :] = scale
            aff_ref[1:2, :] = be_ref[...] - m * scale

    return pl.pallas_call(
        body,
        grid=(nblk,),
        in_specs=[
            pl.BlockSpec((blk, D), lambda i: (i, 0)),
            pl.BlockSpec((blk, D), lambda i: (i, 0)),
            pl.BlockSpec((blk, D), lambda i: (i, 0)),
            pl.BlockSpec((D, hmid), lambda i: (0, 0)),
            pl.BlockSpec((1, hmid), lambda i: (0, 0)),
            pl.BlockSpec((hmid, h2), lambda i: (0, 0)),
            pl.BlockSpec((1, h2), lambda i: (0, 0)),
            pl.BlockSpec((1, h2), lambda i: (0, 0)),
            pl.BlockSpec((1, h2), lambda i: (0, 0)),
        ],
        out_specs=[
            pl.BlockSpec((blk, h2), lambda i: (i, 0)),
            pl.BlockSpec((2, h2), lambda i: (0, 0)),
        ],
        out_shape=[
            jax.ShapeDtypeStruct((N, h2), jnp.float32),
            jax.ShapeDtypeStruct((2, h2), jnp.float32),
        ],
        scratch_shapes=[
            pltpu.VMEM((1, h2), jnp.float32),
            pltpu.VMEM((1, h2), jnp.float32),
        ],
    )(x, parts[0], parts[1], W1, b1.reshape(1, -1), W2, b2.reshape(1, -1),
      g.reshape(1, -1), be.reshape(1, -1))


def _tc_bn_pool(y, aff, batch3, num_graphs, blk=1000):
    N, w = y.shape
    nblk = N // blk

    def body(y_ref, aff_ref, b_ref, h_ref, pool_ref, acc_ref):
        i = pl.program_id(0)
        h = jnp.maximum(y_ref[...] * aff_ref[0:1, :] + aff_ref[1:2, :], 0.0)
        h_ref[...] = h
        bt = b_ref[0]
        onehot = (lax.broadcasted_iota(jnp.int32, (num_graphs, blk), 0)
                  == bt).astype(jnp.float32)

        @pl.when(i == 0)
        def _():
            acc_ref[...] = jnp.zeros_like(acc_ref)

        acc_ref[...] += jnp.dot(onehot, h,
                                preferred_element_type=jnp.float32, precision=lax.Precision.HIGHEST)

        @pl.when(i == nblk - 1)
        def _():
            pool_ref[...] = acc_ref[...]

    return pl.pallas_call(
        body,
        grid=(nblk,),
        in_specs=[
            pl.BlockSpec((blk, w), lambda i: (i, 0)),
            pl.BlockSpec((2, w), lambda i: (0, 0)),
            pl.BlockSpec((1, 1, blk), lambda i: (i, 0, 0)),
        ],
        out_specs=[
            pl.BlockSpec((blk, w), lambda i: (i, 0)),
            pl.BlockSpec((num_graphs, w), lambda i: (0, 0)),
        ],
        out_shape=[
            jax.ShapeDtypeStruct((N, w), jnp.float32),
            jax.ShapeDtypeStruct((num_graphs, w), jnp.float32),
        ],
        scratch_shapes=[pltpu.VMEM((num_graphs, w), jnp.float32)],
    )(y, aff, batch3)


def _tc_readout(hcat, oW1, ob1, og, obe, oW3, ob3, cblk=128):
    gn, rd = hcat.shape
    rh = oW1.shape[1]
    ncls = oW3.shape[1]
    nblk = rh // cblk

    def body(hc_ref, w1_ref, b1_ref, g_ref, be_ref, w3_ref, b3_ref,
             out_ref, acc_ref):
        i = pl.program_id(0)
        t = (jnp.dot(hc_ref[...].astype(jnp.bfloat16),
                     w1_ref[...].astype(jnp.bfloat16),
                     preferred_element_type=jnp.float32) + b1_ref[...])
        m = jnp.mean(t, axis=0, keepdims=True)
        tc = t - m
        v = jnp.mean(tc * tc, axis=0, keepdims=True)
        h = jnp.maximum(
            g_ref[...] * (t - m) / jnp.sqrt(v + 1e-5) + be_ref[...], 0.0)

        @pl.when(i == 0)
        def _():
            acc_ref[...] = jnp.zeros_like(acc_ref)

        acc_ref[...] += jnp.dot(h.astype(jnp.bfloat16),
                                w3_ref[...].astype(jnp.bfloat16),
                                preferred_element_type=jnp.float32)

        @pl.when(i == nblk - 1)
        def _():
            out_ref[...] = acc_ref[...] + b3_ref[...]

    return pl.pallas_call(
        body,
        grid=(nblk,),
        in_specs=[
            pl.BlockSpec((gn, rd), lambda i: (0, 0)),
            pl.BlockSpec((rd, cblk), lambda i: (0, i)),
            pl.BlockSpec((1, cblk), lambda i: (0, i)),
            pl.BlockSpec((1, cblk), lambda i: (0, i)),
            pl.BlockSpec((1, cblk), lambda i: (0, i)),
            pl.BlockSpec((cblk, ncls), lambda i: (i, 0)),
            pl.BlockSpec((1, ncls), lambda i: (0, 0)),
        ],
        out_specs=pl.BlockSpec((gn, ncls), lambda i: (0, 0)),
        out_shape=jax.ShapeDtypeStruct((gn, ncls), jnp.float32),
        scratch_shapes=[pltpu.VMEM((gn, ncls), jnp.float32)],
    )(hcat, oW1, ob1.reshape(1, -1), og.reshape(1, -1), obe.reshape(1, -1),
      oW3, ob3.reshape(1, -1))


def kernel(x, edge_index, batch, fingerprint, c1W1, c1b1, c1W2, c1b2, g1,
           be1, c2W1, c2b1, c2W2, c2b2, g2, be2, c3W1, c3b1, c3W2, c3b2,
           g3, be3, oW1, ob1, og, obe, oW3, ob3):
    N, D = x.shape
    E = edge_index.shape[1]
    G = fingerprint.shape[0]

    n_pad = ((N + _NS * 64 - 1) // (_NS * 64)) * (_NS * 64)
    per_w = -(-E // (_NW * _CHUNK)) * _CHUNK
    e_pad = per_w * _NW
    nch = per_w // _CHUNK

    src = edge_index[0]
    dst = edge_index[1]
    pad = e_pad - E
    src3 = jnp.concatenate(
        [src, jnp.zeros((pad,), jnp.int32)]).reshape(_NW, nch, _CHUNK)
    dst3 = jnp.concatenate(
        [dst, jnp.full((pad,), N, jnp.int32)]).reshape(_NW, nch, _CHUNK)
    zrows = jnp.zeros((n_pad // _NS, D), jnp.float32)
    batch3 = batch.reshape(10, 1, N // 10)

    agg1 = _sc_segsum(x, src3, dst3, zrows, n_pad)
    y1, aff1 = _tc_mlp_stats(x, agg1, c1W1, c1b1, c1W2, c1b2, g1, be1)
    h1, p1 = _tc_bn_pool(y1, aff1, batch3, G)

    agg2 = _sc_segsum(h1, src3, dst3, zrows, n_pad)
    y2, aff2 = _tc_mlp_stats(h1, agg2, c2W1, c2b1, c2W2, c2b2, g2, be2)
    h2, p2 = _tc_bn_pool(y2, aff2, batch3, G)

    agg3 = _sc_segsum(h2, src3, dst3, zrows, n_pad)
    y3, aff3 = _tc_mlp_stats(h2, agg3, c3W1, c3b1, c3W2, c3b2, g3, be3)
    _h3, p3 = _tc_bn_pool(y3, aff3, batch3, G)

    hcat = jnp.concatenate([p1, p2, p3, fingerprint], axis=1)
    return _tc_readout(hcat, oW1, ob1, og, obe, oW3, ob3)
